# baseline (speedup 1.0000x reference)
import functools

import jax
import jax.numpy as jnp
from jax import lax
from jax.experimental import pallas as pl
from jax.experimental.pallas import tpu as pltpu
from jax.experimental.pallas import tpu_sc as plsc

VOCAB = 100000
EMB = 32
BT = 1024
VT = 2048
NV = (VOCAB + VT - 1) // VT
EDGE = VOCAB - (NV - 1) * VT
NBUF = 2
NSPLIT = 4
RS = BT // NSPLIT


def _make_sc_gather(V, D, B):
    info = plsc.get_sparse_core_info()
    nc, ns = info.num_cores, info.num_subcores
    nw = nc * ns
    b_per_w = B // nw
    mesh = plsc.VectorSubcoreMesh(core_axis_name="c", subcore_axis_name="s")

    @functools.partial(
        pl.kernel,
        mesh=mesh,
        compiler_params=pltpu.CompilerParams(use_tc_tiling_on_sc=False),
        out_type=jax.ShapeDtypeStruct((B, D), jnp.float32),
        scratch_types=[
            pltpu.VMEM((b_per_w,), jnp.int32),
            pltpu.VMEM((b_per_w, D), jnp.float32),
            pltpu.SemaphoreType.DMA,
        ],
    )
    def gather(table_hbm, idx_hbm, out_hbm, idx_v, rows_v, sem):
        wid = lax.axis_index("s") * nc + lax.axis_index("c")
        base = wid * b_per_w
        pltpu.sync_copy(idx_hbm.at[pl.ds(base, b_per_w)], idx_v)
        pltpu.async_copy(table_hbm.at[idx_v], rows_v, sem).wait()
        pltpu.sync_copy(rows_v, out_hbm.at[pl.ds(base, b_per_w)])

    return gather


def _copies(j, buf_ref, out_ref, sem_ref):
    slot = lax.rem(j, NBUF)
    return [
        pltpu.make_async_copy(
            buf_ref.at[slot, pl.ds(k * RS, RS), :],
            out_ref.at[pl.ds(k * RS, RS), pl.ds(j * VT, VT)],
            sem_ref.at[slot, k])
        for k in range(NSPLIT)
    ]


def _edge_copy(ebuf_ref, out_ref, esem_ref):
    return pltpu.make_async_copy(
        ebuf_ref,
        out_ref.at[:, pl.ds((NV - 1) * VT, EDGE)],
        esem_ref)


def _logits_loss_body(emb_ref, w_ref, b_ref, t_ref, out_ref, loss_ref,
                      buf_ref, ebuf_ref, m_ref, s_ref, g_ref,
                      sem_ref, esem_ref):
    j = pl.program_id(0)
    slot = lax.rem(j, NBUF)

    @pl.when(j == 0)
    def _init():
        m_ref[...] = jnp.full_like(m_ref, -jnp.inf)
        s_ref[...] = jnp.zeros_like(s_ref)
        g_ref[...] = jnp.zeros_like(g_ref)

    x = jnp.dot(emb_ref[...], w_ref[...],
                preferred_element_type=jnp.float32) + b_ref[...]

    @pl.when(j >= NBUF)
    def _wait_prev():
        for cp in _copies(j - NBUF, buf_ref, out_ref, sem_ref):
            cp.wait()

    @pl.when(j < NV - 1)
    def _issue_full():
        buf_ref[slot] = x
        for cp in _copies(j, buf_ref, out_ref, sem_ref):
            cp.start()

    @pl.when(j == NV - 1)
    def _issue_edge():
        ebuf_ref[...] = x[:, :EDGE]
        _edge_copy(ebuf_ref, out_ref, esem_ref).start()

    col = j * VT + lax.broadcasted_iota(jnp.int32, (BT, VT), 1)
    xm = jnp.where(col < VOCAB, x, -jnp.inf)
    m_old = m_ref[...]
    m_new = jnp.maximum(m_old, jnp.max(xm, axis=1, keepdims=True))
    s_ref[...] = (s_ref[...] * jnp.exp(m_old - m_new)
                  + jnp.sum(jnp.exp(xm - m_new), axis=1, keepdims=True))
    m_ref[...] = m_new
    g_ref[...] += jnp.sum(jnp.where(col == t_ref[...], x, 0.0),
                          axis=1, keepdims=True)

    @pl.when(j == NV - 1)
    def _fin():
        for cp in _copies(j - 1, buf_ref, out_ref, sem_ref):
            cp.wait()
        _edge_copy(ebuf_ref, out_ref, esem_ref).wait()
        nll = m_ref[...] + jnp.log(s_ref[...]) - g_ref[...]
        loss_ref[0, 0] = jnp.sum(nll) * (1.0 / BT)


def _logits_and_loss(emb, W, b2, tflat):
    return pl.pallas_call(
        _logits_loss_body,
        grid=(NV,),
        in_specs=[
            pl.BlockSpec((BT, EMB), lambda j: (0, 0)),
            pl.BlockSpec((EMB, VT), lambda j: (0, j)),
            pl.BlockSpec((1, VT), lambda j: (0, j)),
            pl.BlockSpec((BT, 1), lambda j: (0, 0)),
        ],
        out_specs=[
            pl.BlockSpec(memory_space=pl.ANY),
            pl.BlockSpec(memory_space=pltpu.SMEM),
        ],
        out_shape=[
            jax.ShapeDtypeStruct((BT, VOCAB), jnp.float32),
            jax.ShapeDtypeStruct((1, 1), jnp.float32),
        ],
        scratch_shapes=[
            pltpu.VMEM((NBUF, BT, VT), jnp.float32),
            pltpu.VMEM((BT, EDGE), jnp.float32),
            pltpu.VMEM((BT, 1), jnp.float32),
            pltpu.VMEM((BT, 1), jnp.float32),
            pltpu.VMEM((BT, 1), jnp.float32),
            pltpu.SemaphoreType.DMA((NBUF, NSPLIT)),
            pltpu.SemaphoreType.DMA,
        ],
    )(emb, W, b2, tflat)


_sc_gather_cache = []


def _sc_gather(table, idx_flat):
    if not _sc_gather_cache:
        _sc_gather_cache.append(_make_sc_gather(VOCAB, EMB, BT))
    return _sc_gather_cache[0](table, idx_flat)


def kernel(idx, targets, token_table, W, b):
    idx_flat = idx.reshape(BT).astype(jnp.int32)
    tflat = targets.reshape(BT, 1).astype(jnp.int32)
    emb = _sc_gather(token_table, idx_flat)
    logits, loss = _logits_and_loss(emb, W, b.reshape(1, VOCAB), tflat)
    return logits, loss[0, 0]
